# baseline (speedup 1.0000x reference)
import functools

import jax
import jax.numpy as jnp
from jax import lax
from jax.experimental import pallas as pl
from jax.experimental.pallas import tpu as pltpu
from jax.experimental.pallas import tpu_sc as plsc

N = 10000
D = 256
HALF = 128
NC = 2
NS = 16
EB = 128
NBT = 80
NBTA = 81
E_PAD = NS * EB * NBT
E_PADA = NS * EB * NBTA
JUNK = N
RPT = 632
ACC_ROWS = 10008
LAST = N - (NS - 1) * RPT

_MESH = plsc.VectorSubcoreMesh(
    core_axis_name="c", subcore_axis_name="s", num_cores=NC, num_subcores=NS)


def _deg_body(dst3, ones_in, zeros_in, degp0, degp1, didx_v, ones_v, deg_sp):
    c = lax.axis_index("c")
    t = lax.axis_index("s")

    @pl.when(t < NS - 1)
    def _():
        pltpu.sync_copy(zeros_in, deg_sp.at[pl.ds(t * RPT, RPT)])

    @pl.when(t == NS - 1)
    def _():
        pltpu.sync_copy(zeros_in.at[pl.ds(0, LAST)],
                        deg_sp.at[pl.ds((NS - 1) * RPT, LAST)])

    pltpu.sync_copy(ones_in, ones_v)
    pltpu.sync_copy(dst3.at[t, pl.ds(c * (NBT // NC), NBT // NC)], didx_v)
    plsc.subcore_barrier()

    @pl.loop(0, NBT // NC)
    def _(b):
        pltpu.sync_copy(ones_v, deg_sp.at[didx_v.at[b]], add=True)

    plsc.subcore_barrier()

    def copy_out(dst_ref):
        @pl.when(t < NS - 1)
        def _():
            pltpu.sync_copy(deg_sp.at[pl.ds(t * RPT, RPT)],
                            dst_ref.at[pl.ds(t * RPT, RPT)])

        @pl.when(t == NS - 1)
        def _():
            pltpu.sync_copy(deg_sp.at[pl.ds((NS - 1) * RPT, LAST)],
                            dst_ref.at[pl.ds((NS - 1) * RPT, LAST)])

    @pl.when(c == 0)
    def _():
        copy_out(degp0)

    @pl.when(c == 1)
    def _():
        copy_out(degp1)


_deg_call = pl.kernel(
    _deg_body,
    out_type=[jax.ShapeDtypeStruct((N, HALF), jnp.float32)] * 2,
    mesh=_MESH,
    scratch_types=[
        pltpu.VMEM((NBT // NC, EB), jnp.int32),
        pltpu.VMEM((EB, HALF), jnp.float32),
        pltpu.VMEM_SHARED((ACC_ROWS, HALF), jnp.float32),
    ],
)


NBUF = 3
NWIN = NBTA // NBUF


WB = NBUF * EB


def _agg_body(src1, dst4, ht0, ht1, out0, out1, sidx_1, didx_w,
              b0, b1, b2, s0, s1, s2, sd, acc_sp):
    bufs = (b0, b1, b2)
    sems = (s0, s1, s2)
    c = lax.axis_index("c")
    t = lax.axis_index("s")

    def load_src_win(w, p):
        pltpu.sync_copy(src1.at[pl.ds(t * (NWIN * WB) + w * WB, WB)],
                        sidx_1.at[pl.ds(p * WB, WB)])

    def one_side(tbl, out):
        @pl.when(t < NS - 1)
        def _():
            pltpu.sync_copy(tbl.at[pl.ds(t * RPT, RPT)],
                            acc_sp.at[pl.ds(t * RPT, RPT)])

        @pl.when(t == NS - 1)
        def _():
            pltpu.sync_copy(tbl.at[pl.ds((NS - 1) * RPT, LAST)],
                            acc_sp.at[pl.ds((NS - 1) * RPT, LAST)])

        plsc.subcore_barrier()

        load_src_win(0, 0)
        pltpu.async_copy(dst4.at[t, 0], didx_w.at[pl.ds(0, NBUF)], sd)
        for j in range(NBUF):
            pltpu.async_copy(
                tbl.at[sidx_1.at[pl.ds(j * EB, EB)]], bufs[j], sems[j])

        @pl.loop(0, NWIN)
        def _(w):
            p = lax.rem(w, 2)
            q = 1 - p

            @pl.when(w + 1 < NWIN)
            def _():
                load_src_win(w + 1, q)

            for k in range(NBUF):
                pltpu.make_async_copy(
                    tbl.at[pl.ds(0, EB)], bufs[k], sems[k]).wait()
                if k == 0:
                    pltpu.make_async_copy(
                        dst4.at[t, 0], didx_w.at[pl.ds(0, NBUF)], sd).wait()
                pltpu.sync_copy(bufs[k], acc_sp.at[didx_w.at[k]], add=True)

                @pl.when(w + 1 < NWIN)
                def _():
                    pltpu.async_copy(
                        tbl.at[sidx_1.at[pl.ds(q * WB + k * EB, EB)]],
                        bufs[k], sems[k])

            @pl.when(w + 1 < NWIN)
            def _():
                pltpu.async_copy(
                    dst4.at[t, w + 1], didx_w.at[pl.ds(0, NBUF)], sd)

        plsc.subcore_barrier()

        @pl.when(t < NS - 1)
        def _():
            pltpu.sync_copy(acc_sp.at[pl.ds(t * RPT, RPT)],
                            out.at[pl.ds(t * RPT, RPT)])

        @pl.when(t == NS - 1)
        def _():
            pltpu.sync_copy(acc_sp.at[pl.ds((NS - 1) * RPT, LAST)],
                            out.at[pl.ds((NS - 1) * RPT, LAST)])

    @pl.when(c == 0)
    def _():
        one_side(ht0, out0)

    @pl.when(c == 1)
    def _():
        one_side(ht1, out1)


_agg_call = pl.kernel(
    _agg_body,
    out_type=[jax.ShapeDtypeStruct((N, HALF), jnp.float32)] * 2,
    mesh=_MESH,
    scratch_types=[
        pltpu.VMEM((2 * WB,), jnp.int32),
        pltpu.VMEM((8, EB), jnp.int32),
    ] + [pltpu.VMEM((EB, HALF), jnp.float32)] * NBUF
      + [pltpu.SemaphoreType.DMA] * (NBUF + 1)
      + [pltpu.VMEM_SHARED((ACC_ROWS, HALF), jnp.float32)],
)


_RB = 2000


def _mm1_body(x_ref, w_ref, d0_ref, d1_ref, ht0_ref, ht1_ref, dis_ref):
    deg = d0_ref[:, :1] + d1_ref[:, :1] + 1.0
    dis = lax.rsqrt(deg)
    h = jnp.dot(x_ref[...], w_ref[...], preferred_element_type=jnp.float32)
    ht = h * dis
    ht0_ref[...] = ht[:, :HALF]
    ht1_ref[...] = ht[:, HALF:]
    dis_ref[...] = dis


def _mm2_body(a0_ref, a1_ref, dis_ref, b_ref, w_ref, ht0_ref, ht1_ref):
    acc = jnp.concatenate([a0_ref[...], a1_ref[...]], axis=1)
    dis = dis_ref[...]
    z = jnp.maximum(acc * dis + b_ref[...], 0.0)
    h = jnp.dot(z, w_ref[...], preferred_element_type=jnp.float32)
    ht = h * dis
    ht0_ref[...] = ht[:, :HALF]
    ht1_ref[...] = ht[:, HALF:]


def _fin_body(a0_ref, a1_ref, dis_ref, b_ref, o_ref):
    acc = jnp.concatenate([a0_ref[...], a1_ref[...]], axis=1)
    o_ref[...] = acc * dis_ref[...] + b_ref[...]


def _rows(shape):
    return pl.BlockSpec((_RB,) + shape[1:], lambda i: (i, 0))


def _whole(shape):
    return pl.BlockSpec(shape, lambda i: (0, 0))


_mm1_call = pl.pallas_call(
    _mm1_body,
    grid=(N // _RB,),
    in_specs=[_rows((N, D)), _whole((D, D)), _rows((N, HALF)), _rows((N, HALF))],
    out_specs=[_rows((N, HALF)), _rows((N, HALF)), _rows((N, 1))],
    out_shape=[jax.ShapeDtypeStruct((N, HALF), jnp.float32),
               jax.ShapeDtypeStruct((N, HALF), jnp.float32),
               jax.ShapeDtypeStruct((N, 1), jnp.float32)],
)

_mm2_call = pl.pallas_call(
    _mm2_body,
    grid=(N // _RB,),
    in_specs=[_rows((N, HALF)), _rows((N, HALF)), _rows((N, 1)),
              _whole((1, D)), _whole((D, D))],
    out_specs=[_rows((N, HALF)), _rows((N, HALF))],
    out_shape=[jax.ShapeDtypeStruct((N, HALF), jnp.float32),
               jax.ShapeDtypeStruct((N, HALF), jnp.float32)],
)

_fin_call = pl.pallas_call(
    _fin_body,
    grid=(N // _RB,),
    in_specs=[_rows((N, HALF)), _rows((N, HALF)), _rows((N, 1)),
              _whole((1, D))],
    out_specs=_rows((N, D)),
    out_shape=jax.ShapeDtypeStruct((N, D), jnp.float32),
)


def kernel(x, edge_index, W1, b1, W2, b2):
    src = edge_index[0].astype(jnp.int32)
    dst = edge_index[1].astype(jnp.int32)
    npad = E_PAD - src.shape[0]
    npada = E_PADA - src.shape[0]
    dst3 = jnp.concatenate(
        [dst, jnp.full((npad,), JUNK, jnp.int32)]).reshape(NS, NBT, EB)
    src1 = jnp.concatenate([src, jnp.zeros((npada,), jnp.int32)])
    dst4 = jnp.concatenate(
        [dst, jnp.full((npada,), JUNK, jnp.int32)]).reshape(NS, NWIN, NBUF, EB)
    ones_in = jnp.ones((EB, HALF), jnp.float32)
    zeros_in = jnp.zeros((RPT, HALF), jnp.float32)

    degp0, degp1 = _deg_call(dst3, ones_in, zeros_in)
    ht0, ht1, dis = _mm1_call(x, W1, degp0, degp1)
    a10, a11 = _agg_call(src1, dst4, ht0, ht1)
    h20, h21 = _mm2_call(a10, a11, dis, b1.reshape(1, D), W2)
    a20, a21 = _agg_call(src1, dst4, h20, h21)
    return _fin_call(a20, a21, dis, b2.reshape(1, D))


# speedup vs baseline: 1.1928x; 1.1928x over previous
import functools

import jax
import jax.numpy as jnp
from jax import lax
from jax.experimental import pallas as pl
from jax.experimental.pallas import tpu as pltpu
from jax.experimental.pallas import tpu_sc as plsc

N = 10000
D = 256
HALF = 128
NC = 2
NS = 16
EB = 128
NBT = 80
E_PAD = NS * EB * NBT
JUNK = N
RPT = 632
ACC_ROWS = 10008
LAST = N - (NS - 1) * RPT

_MESH = plsc.VectorSubcoreMesh(
    core_axis_name="c", subcore_axis_name="s", num_cores=NC, num_subcores=NS)


def _deg_body(dst3, ones_in, zeros_in, degp0, degp1, didx_v, ones_v, deg_sp):
    c = lax.axis_index("c")
    t = lax.axis_index("s")

    @pl.when(t < NS - 1)
    def _():
        pltpu.sync_copy(zeros_in, deg_sp.at[pl.ds(t * RPT, RPT)])

    @pl.when(t == NS - 1)
    def _():
        pltpu.sync_copy(zeros_in.at[pl.ds(0, LAST)],
                        deg_sp.at[pl.ds((NS - 1) * RPT, LAST)])

    pltpu.sync_copy(ones_in, ones_v)
    pltpu.sync_copy(dst3.at[t, pl.ds(c * (NBT // NC), NBT // NC)], didx_v)
    plsc.subcore_barrier()

    @pl.loop(0, NBT // NC)
    def _(b):
        pltpu.sync_copy(ones_v, deg_sp.at[didx_v.at[b]], add=True)

    plsc.subcore_barrier()

    def copy_out(dst_ref):
        @pl.when(t < NS - 1)
        def _():
            pltpu.sync_copy(deg_sp.at[pl.ds(t * RPT, RPT)],
                            dst_ref.at[pl.ds(t * RPT, RPT)])

        @pl.when(t == NS - 1)
        def _():
            pltpu.sync_copy(deg_sp.at[pl.ds((NS - 1) * RPT, LAST)],
                            dst_ref.at[pl.ds((NS - 1) * RPT, LAST)])

    @pl.when(c == 0)
    def _():
        copy_out(degp0)

    @pl.when(c == 1)
    def _():
        copy_out(degp1)


_deg_call = pl.kernel(
    _deg_body,
    out_type=[jax.ShapeDtypeStruct((N, HALF), jnp.float32)] * 2,
    mesh=_MESH,
    scratch_types=[
        pltpu.VMEM((NBT // NC, EB), jnp.int32),
        pltpu.VMEM((EB, HALF), jnp.float32),
        pltpu.VMEM_SHARED((ACC_ROWS, HALF), jnp.float32),
    ],
)


NBUF = 2
CH = 8
NWIN = NBT // CH


def _agg_body(src3, dst3, ht0, ht1, out0, out1, sidx_w, didx_w,
              b0, b1, s0, s1, acc_sp):
    bufs = (b0, b1)
    sems = (s0, s1)
    c = lax.axis_index("c")
    t = lax.axis_index("s")

    def load_win(w, p):
        pltpu.sync_copy(src3.at[t, pl.ds(w * CH, CH)],
                        sidx_w.at[pl.ds(p * CH, CH)])
        pltpu.sync_copy(dst3.at[t, pl.ds(w * CH, CH)],
                        didx_w.at[pl.ds(p * CH, CH)])

    def one_side(tbl, out):
        @pl.when(t < NS - 1)
        def _():
            pltpu.sync_copy(tbl.at[pl.ds(t * RPT, RPT)],
                            acc_sp.at[pl.ds(t * RPT, RPT)])

        @pl.when(t == NS - 1)
        def _():
            pltpu.sync_copy(tbl.at[pl.ds((NS - 1) * RPT, LAST)],
                            acc_sp.at[pl.ds((NS - 1) * RPT, LAST)])

        plsc.subcore_barrier()

        load_win(0, 0)
        for j in range(NBUF):
            pltpu.async_copy(tbl.at[sidx_w.at[j]], bufs[j], sems[j])

        @pl.loop(0, NWIN)
        def _(w):
            p = lax.rem(w, 2)
            q = 1 - p

            @pl.when(w + 1 < NWIN)
            def _():
                load_win(w + 1, q)

            for k in range(CH):
                j = k % NBUF
                pltpu.make_async_copy(
                    tbl.at[pl.ds(0, EB)], bufs[j], sems[j]).wait()
                pltpu.sync_copy(bufs[j], acc_sp.at[didx_w.at[p * CH + k]],
                                add=True)
                if k < CH - NBUF:
                    pltpu.async_copy(
                        tbl.at[sidx_w.at[p * CH + k + NBUF]], bufs[j], sems[j])
                else:
                    @pl.when(w + 1 < NWIN)
                    def _():
                        pltpu.async_copy(
                            tbl.at[sidx_w.at[q * CH + k + NBUF - CH]],
                            bufs[j], sems[j])

        plsc.subcore_barrier()

        @pl.when(t < NS - 1)
        def _():
            pltpu.sync_copy(acc_sp.at[pl.ds(t * RPT, RPT)],
                            out.at[pl.ds(t * RPT, RPT)])

        @pl.when(t == NS - 1)
        def _():
            pltpu.sync_copy(acc_sp.at[pl.ds((NS - 1) * RPT, LAST)],
                            out.at[pl.ds((NS - 1) * RPT, LAST)])

    @pl.when(c == 0)
    def _():
        one_side(ht0, out0)

    @pl.when(c == 1)
    def _():
        one_side(ht1, out1)


_agg_call = pl.kernel(
    _agg_body,
    out_type=[jax.ShapeDtypeStruct((N, HALF), jnp.float32)] * 2,
    mesh=_MESH,
    scratch_types=[
        pltpu.VMEM((2 * CH, EB), jnp.int32),
        pltpu.VMEM((2 * CH, EB), jnp.int32),
    ] + [pltpu.VMEM((EB, HALF), jnp.float32)] * NBUF
      + [pltpu.SemaphoreType.DMA] * NBUF
      + [pltpu.VMEM_SHARED((ACC_ROWS, HALF), jnp.float32)],
)


_RB = 2000


def _mm1_body(x_ref, w_ref, d0_ref, d1_ref, ht0_ref, ht1_ref, dis_ref):
    deg = d0_ref[:, :1] + d1_ref[:, :1] + 1.0
    dis = lax.rsqrt(deg)
    h = jnp.dot(x_ref[...], w_ref[...], preferred_element_type=jnp.float32)
    ht = h * dis
    ht0_ref[...] = ht[:, :HALF]
    ht1_ref[...] = ht[:, HALF:]
    dis_ref[...] = dis


def _mm2_body(a0_ref, a1_ref, dis_ref, b_ref, w_ref, ht0_ref, ht1_ref):
    acc = jnp.concatenate([a0_ref[...], a1_ref[...]], axis=1)
    dis = dis_ref[...]
    z = jnp.maximum(acc * dis + b_ref[...], 0.0)
    h = jnp.dot(z, w_ref[...], preferred_element_type=jnp.float32)
    ht = h * dis
    ht0_ref[...] = ht[:, :HALF]
    ht1_ref[...] = ht[:, HALF:]


def _fin_body(a0_ref, a1_ref, dis_ref, b_ref, o_ref):
    acc = jnp.concatenate([a0_ref[...], a1_ref[...]], axis=1)
    o_ref[...] = acc * dis_ref[...] + b_ref[...]


def _rows(shape):
    return pl.BlockSpec((_RB,) + shape[1:], lambda i: (i, 0))


def _whole(shape):
    return pl.BlockSpec(shape, lambda i: (0, 0))


_mm1_call = pl.pallas_call(
    _mm1_body,
    grid=(N // _RB,),
    in_specs=[_rows((N, D)), _whole((D, D)), _rows((N, HALF)), _rows((N, HALF))],
    out_specs=[_rows((N, HALF)), _rows((N, HALF)), _rows((N, 1))],
    out_shape=[jax.ShapeDtypeStruct((N, HALF), jnp.float32),
               jax.ShapeDtypeStruct((N, HALF), jnp.float32),
               jax.ShapeDtypeStruct((N, 1), jnp.float32)],
)

_mm2_call = pl.pallas_call(
    _mm2_body,
    grid=(N // _RB,),
    in_specs=[_rows((N, HALF)), _rows((N, HALF)), _rows((N, 1)),
              _whole((1, D)), _whole((D, D))],
    out_specs=[_rows((N, HALF)), _rows((N, HALF))],
    out_shape=[jax.ShapeDtypeStruct((N, HALF), jnp.float32),
               jax.ShapeDtypeStruct((N, HALF), jnp.float32)],
)

_fin_call = pl.pallas_call(
    _fin_body,
    grid=(N // _RB,),
    in_specs=[_rows((N, HALF)), _rows((N, HALF)), _rows((N, 1)),
              _whole((1, D))],
    out_specs=_rows((N, D)),
    out_shape=jax.ShapeDtypeStruct((N, D), jnp.float32),
)


def kernel(x, edge_index, W1, b1, W2, b2):
    src = edge_index[0].astype(jnp.int32)
    dst = edge_index[1].astype(jnp.int32)
    npad = E_PAD - src.shape[0]
    src3 = jnp.concatenate(
        [src, jnp.zeros((npad,), jnp.int32)]).reshape(NS, NBT, EB)
    dst3 = jnp.concatenate(
        [dst, jnp.full((npad,), JUNK, jnp.int32)]).reshape(NS, NBT, EB)
    ones_in = jnp.ones((EB, HALF), jnp.float32)
    zeros_in = jnp.zeros((RPT, HALF), jnp.float32)

    degp0, degp1 = _deg_call(dst3, ones_in, zeros_in)
    ht0, ht1, dis = _mm1_call(x, W1, degp0, degp1)
    a10, a11 = _agg_call(src3, dst3, ht0, ht1)
    h20, h21 = _mm2_call(a10, a11, dis, b1.reshape(1, D), W2)
    a20, a21 = _agg_call(src3, dst3, h20, h21)
    return _fin_call(a20, a21, dis, b2.reshape(1, D))


# speedup vs baseline: 1.2037x; 1.0092x over previous
import functools

import jax
import jax.numpy as jnp
from jax import lax
from jax.experimental import pallas as pl
from jax.experimental.pallas import tpu as pltpu
from jax.experimental.pallas import tpu_sc as plsc

N = 10000
D = 256
HALF = 128
NC = 2
NS = 16
EB = 128
NBT = 80
E_PAD = NS * EB * NBT
JUNK = N
RPT = 632
ACC_ROWS = 10008
LAST = N - (NS - 1) * RPT

_MESH = plsc.VectorSubcoreMesh(
    core_axis_name="c", subcore_axis_name="s", num_cores=NC, num_subcores=NS)


def _deg_body(dst3, ones_in, zeros_in, degp0, degp1, didx_v, ones_v, deg_sp):
    c = lax.axis_index("c")
    t = lax.axis_index("s")

    @pl.when(t < NS - 1)
    def _():
        pltpu.sync_copy(zeros_in, deg_sp.at[pl.ds(t * RPT, RPT)])

    @pl.when(t == NS - 1)
    def _():
        pltpu.sync_copy(zeros_in.at[pl.ds(0, LAST)],
                        deg_sp.at[pl.ds((NS - 1) * RPT, LAST)])

    pltpu.sync_copy(ones_in, ones_v)
    pltpu.sync_copy(dst3.at[t, pl.ds(c * (NBT // NC), NBT // NC)], didx_v)
    plsc.subcore_barrier()

    @pl.loop(0, NBT // NC)
    def _(b):
        pltpu.sync_copy(ones_v, deg_sp.at[didx_v.at[b]], add=True)

    plsc.subcore_barrier()

    def copy_out(dst_ref):
        @pl.when(t < NS - 1)
        def _():
            pltpu.sync_copy(deg_sp.at[pl.ds(t * RPT, RPT)],
                            dst_ref.at[pl.ds(t * RPT, RPT)])

        @pl.when(t == NS - 1)
        def _():
            pltpu.sync_copy(deg_sp.at[pl.ds((NS - 1) * RPT, LAST)],
                            dst_ref.at[pl.ds((NS - 1) * RPT, LAST)])

    @pl.when(c == 0)
    def _():
        copy_out(degp0)

    @pl.when(c == 1)
    def _():
        copy_out(degp1)


_deg_call = pl.kernel(
    _deg_body,
    out_type=[jax.ShapeDtypeStruct((N, HALF), jnp.float32)] * 2,
    mesh=_MESH,
    scratch_types=[
        pltpu.VMEM((NBT // NC, EB), jnp.int32),
        pltpu.VMEM((EB, HALF), jnp.float32),
        pltpu.VMEM_SHARED((ACC_ROWS, HALF), jnp.float32),
    ],
)


NBUF = 2
CH = 8
NWIN = NBT // CH


def _agg_body(src3, dst3, ht0, ht1, out0, out1, sidx_w, didx_w,
              b0, b1, s0, s1, acc_sp):
    bufs = (b0, b1)
    sems = (s0, s1)
    c = lax.axis_index("c")
    t = lax.axis_index("s")

    def load_win(w, p):
        pltpu.sync_copy(src3.at[t, pl.ds(w * CH, CH)],
                        sidx_w.at[pl.ds(p * CH, CH)])
        pltpu.sync_copy(dst3.at[t, pl.ds(w * CH, CH)],
                        didx_w.at[pl.ds(p * CH, CH)])

    def one_side(tbl, out):
        @pl.when(t < NS - 1)
        def _():
            pltpu.sync_copy(tbl.at[pl.ds(t * RPT, RPT)],
                            acc_sp.at[pl.ds(t * RPT, RPT)])

        @pl.when(t == NS - 1)
        def _():
            pltpu.sync_copy(tbl.at[pl.ds((NS - 1) * RPT, LAST)],
                            acc_sp.at[pl.ds((NS - 1) * RPT, LAST)])

        plsc.subcore_barrier()

        load_win(0, 0)
        for j in range(NBUF):
            pltpu.async_copy(tbl.at[sidx_w.at[j]], bufs[j], sems[j])

        @pl.loop(0, NWIN)
        def _(w):
            p = lax.rem(w, 2)
            q = 1 - p

            @pl.when(w + 1 < NWIN)
            def _():
                load_win(w + 1, q)

            for k in range(CH):
                j = k % NBUF
                pltpu.make_async_copy(
                    tbl.at[pl.ds(0, EB)], bufs[j], sems[j]).wait()
                pltpu.sync_copy(bufs[j], acc_sp.at[didx_w.at[p * CH + k]],
                                add=True)
                if k < CH - NBUF:
                    pltpu.async_copy(
                        tbl.at[sidx_w.at[p * CH + k + NBUF]], bufs[j], sems[j])
                else:
                    @pl.when(w + 1 < NWIN)
                    def _():
                        pltpu.async_copy(
                            tbl.at[sidx_w.at[q * CH + k + NBUF - CH]],
                            bufs[j], sems[j])

        plsc.subcore_barrier()

        @pl.when(t < NS - 1)
        def _():
            pltpu.sync_copy(acc_sp.at[pl.ds(t * RPT, RPT)],
                            out.at[pl.ds(t * RPT, RPT)])

        @pl.when(t == NS - 1)
        def _():
            pltpu.sync_copy(acc_sp.at[pl.ds((NS - 1) * RPT, LAST)],
                            out.at[pl.ds((NS - 1) * RPT, LAST)])

    @pl.when(c == 0)
    def _():
        one_side(ht0, out0)

    @pl.when(c == 1)
    def _():
        one_side(ht1, out1)


_agg_call = pl.kernel(
    _agg_body,
    out_type=[jax.ShapeDtypeStruct((N, HALF), jnp.float32)] * 2,
    mesh=_MESH,
    scratch_types=[
        pltpu.VMEM((2 * CH, EB), jnp.int32),
        pltpu.VMEM((2 * CH, EB), jnp.int32),
    ] + [pltpu.VMEM((EB, HALF), jnp.float32)] * NBUF
      + [pltpu.SemaphoreType.DMA] * NBUF
      + [pltpu.VMEM_SHARED((ACC_ROWS, HALF), jnp.float32)],
)


_RB = 2000


def _mm1a_body(x_ref, w_ref, h_ref):
    h_ref[...] = jnp.dot(x_ref[...], w_ref[...],
                         preferred_element_type=jnp.float32)


def _mm1b_body(h_ref, d0_ref, d1_ref, ht0_ref, ht1_ref, dis_ref):
    deg = d0_ref[:, :1] + d1_ref[:, :1] + 1.0
    dis = lax.rsqrt(deg)
    ht = h_ref[...] * dis
    ht0_ref[...] = ht[:, :HALF]
    ht1_ref[...] = ht[:, HALF:]
    dis_ref[...] = dis


def _mm2_body(a0_ref, a1_ref, dis_ref, b_ref, w_ref, ht0_ref, ht1_ref):
    acc = jnp.concatenate([a0_ref[...], a1_ref[...]], axis=1)
    dis = dis_ref[...]
    z = jnp.maximum(acc * dis + b_ref[...], 0.0)
    h = jnp.dot(z, w_ref[...], preferred_element_type=jnp.float32)
    ht = h * dis
    ht0_ref[...] = ht[:, :HALF]
    ht1_ref[...] = ht[:, HALF:]


def _fin_body(a0_ref, a1_ref, dis_ref, b_ref, o_ref):
    acc = jnp.concatenate([a0_ref[...], a1_ref[...]], axis=1)
    o_ref[...] = acc * dis_ref[...] + b_ref[...]


def _rows(shape):
    return pl.BlockSpec((_RB,) + shape[1:], lambda i: (i, 0))


def _whole(shape):
    return pl.BlockSpec(shape, lambda i: (0, 0))


_mm1a_call = pl.pallas_call(
    _mm1a_body,
    grid=(N // _RB,),
    in_specs=[_rows((N, D)), _whole((D, D))],
    out_specs=_rows((N, D)),
    out_shape=jax.ShapeDtypeStruct((N, D), jnp.float32),
)

_mm1b_call = pl.pallas_call(
    _mm1b_body,
    grid=(N // _RB,),
    in_specs=[_rows((N, D)), _rows((N, HALF)), _rows((N, HALF))],
    out_specs=[_rows((N, HALF)), _rows((N, HALF)), _rows((N, 1))],
    out_shape=[jax.ShapeDtypeStruct((N, HALF), jnp.float32),
               jax.ShapeDtypeStruct((N, HALF), jnp.float32),
               jax.ShapeDtypeStruct((N, 1), jnp.float32)],
)

_mm2_call = pl.pallas_call(
    _mm2_body,
    grid=(N // _RB,),
    in_specs=[_rows((N, HALF)), _rows((N, HALF)), _rows((N, 1)),
              _whole((1, D)), _whole((D, D))],
    out_specs=[_rows((N, HALF)), _rows((N, HALF))],
    out_shape=[jax.ShapeDtypeStruct((N, HALF), jnp.float32),
               jax.ShapeDtypeStruct((N, HALF), jnp.float32)],
)

_fin_call = pl.pallas_call(
    _fin_body,
    grid=(N // _RB,),
    in_specs=[_rows((N, HALF)), _rows((N, HALF)), _rows((N, 1)),
              _whole((1, D))],
    out_specs=_rows((N, D)),
    out_shape=jax.ShapeDtypeStruct((N, D), jnp.float32),
)


def kernel(x, edge_index, W1, b1, W2, b2):
    src = edge_index[0].astype(jnp.int32)
    dst = edge_index[1].astype(jnp.int32)
    npad = E_PAD - src.shape[0]
    src3 = jnp.concatenate(
        [src, jnp.zeros((npad,), jnp.int32)]).reshape(NS, NBT, EB)
    dst3 = jnp.concatenate(
        [dst, jnp.full((npad,), JUNK, jnp.int32)]).reshape(NS, NBT, EB)
    ones_in = jnp.ones((EB, HALF), jnp.float32)
    zeros_in = jnp.zeros((RPT, HALF), jnp.float32)

    h1 = _mm1a_call(x, W1)
    degp0, degp1 = _deg_call(dst3, ones_in, zeros_in)
    ht0, ht1, dis = _mm1b_call(h1, degp0, degp1)
    a10, a11 = _agg_call(src3, dst3, ht0, ht1)
    h20, h21 = _mm2_call(a10, a11, dis, b1.reshape(1, D), W2)
    a20, a21 = _agg_call(src3, dst3, h20, h21)
    return _fin_call(a20, a21, dis, b2.reshape(1, D))


# speedup vs baseline: 1.2247x; 1.0174x over previous
import functools

import jax
import jax.numpy as jnp
from jax import lax
from jax.experimental import pallas as pl
from jax.experimental.pallas import tpu as pltpu
from jax.experimental.pallas import tpu_sc as plsc

N = 10000
D = 256
HALF = 128
NC = 2
NS = 16
EB = 128
NBT = 80
E_PAD = NS * EB * NBT
JUNK = N
RPT = 632
ACC_ROWS = 10008
LAST = N - (NS - 1) * RPT

_MESH = plsc.VectorSubcoreMesh(
    core_axis_name="c", subcore_axis_name="s", num_cores=NC, num_subcores=NS)


def _deg_body(dst3, ones_in, zeros_in, degp0, degp1, didx_v, ones_v, deg_sp):
    c = lax.axis_index("c")
    t = lax.axis_index("s")

    @pl.when(t < NS - 1)
    def _():
        pltpu.sync_copy(zeros_in, deg_sp.at[pl.ds(t * RPT, RPT)])

    @pl.when(t == NS - 1)
    def _():
        pltpu.sync_copy(zeros_in.at[pl.ds(0, LAST)],
                        deg_sp.at[pl.ds((NS - 1) * RPT, LAST)])

    pltpu.sync_copy(ones_in, ones_v)
    pltpu.sync_copy(dst3.at[t, pl.ds(c * (NBT // NC), NBT // NC)], didx_v)
    plsc.subcore_barrier()

    @pl.loop(0, NBT // NC)
    def _(b):
        pltpu.sync_copy(ones_v, deg_sp.at[didx_v.at[b]], add=True)

    plsc.subcore_barrier()

    def copy_out(dst_ref):
        @pl.when(t < NS - 1)
        def _():
            pltpu.sync_copy(deg_sp.at[pl.ds(t * RPT, RPT)],
                            dst_ref.at[pl.ds(t * RPT, RPT)])

        @pl.when(t == NS - 1)
        def _():
            pltpu.sync_copy(deg_sp.at[pl.ds((NS - 1) * RPT, LAST)],
                            dst_ref.at[pl.ds((NS - 1) * RPT, LAST)])

    @pl.when(c == 0)
    def _():
        copy_out(degp0)

    @pl.when(c == 1)
    def _():
        copy_out(degp1)


_deg_call = pl.kernel(
    _deg_body,
    out_type=[jax.ShapeDtypeStruct((N, HALF), jnp.float32)] * 2,
    mesh=_MESH,
    scratch_types=[
        pltpu.VMEM((NBT // NC, EB), jnp.int32),
        pltpu.VMEM((EB, HALF), jnp.float32),
        pltpu.VMEM_SHARED((ACC_ROWS, HALF), jnp.float32),
    ],
)


NBUF = 2
CH = 16
NWIN = NBT // CH


def _agg_body(src3, dst3, ht0, ht1, out0, out1, sidx_w, didx_w,
              b0, b1, s0, s1, acc_sp):
    bufs = (b0, b1)
    sems = (s0, s1)
    c = lax.axis_index("c")
    t = lax.axis_index("s")

    def load_win(w, p):
        pltpu.sync_copy(src3.at[t, pl.ds(w * CH, CH)],
                        sidx_w.at[pl.ds(p * CH, CH)])
        pltpu.sync_copy(dst3.at[t, pl.ds(w * CH, CH)],
                        didx_w.at[pl.ds(p * CH, CH)])

    def one_side(tbl, out):
        @pl.when(t < NS - 1)
        def _():
            pltpu.sync_copy(tbl.at[pl.ds(t * RPT, RPT)],
                            acc_sp.at[pl.ds(t * RPT, RPT)])

        @pl.when(t == NS - 1)
        def _():
            pltpu.sync_copy(tbl.at[pl.ds((NS - 1) * RPT, LAST)],
                            acc_sp.at[pl.ds((NS - 1) * RPT, LAST)])

        plsc.subcore_barrier()

        load_win(0, 0)
        for j in range(NBUF):
            pltpu.async_copy(tbl.at[sidx_w.at[j]], bufs[j], sems[j])

        @pl.loop(0, NWIN)
        def _(w):
            p = lax.rem(w, 2)
            q = 1 - p

            @pl.when(w + 1 < NWIN)
            def _():
                load_win(w + 1, q)

            for k in range(CH):
                j = k % NBUF
                pltpu.make_async_copy(
                    tbl.at[pl.ds(0, EB)], bufs[j], sems[j]).wait()
                pltpu.sync_copy(bufs[j], acc_sp.at[didx_w.at[p * CH + k]],
                                add=True)
                if k < CH - NBUF:
                    pltpu.async_copy(
                        tbl.at[sidx_w.at[p * CH + k + NBUF]], bufs[j], sems[j])
                else:
                    @pl.when(w + 1 < NWIN)
                    def _():
                        pltpu.async_copy(
                            tbl.at[sidx_w.at[q * CH + k + NBUF - CH]],
                            bufs[j], sems[j])

        plsc.subcore_barrier()

        @pl.when(t < NS - 1)
        def _():
            pltpu.sync_copy(acc_sp.at[pl.ds(t * RPT, RPT)],
                            out.at[pl.ds(t * RPT, RPT)])

        @pl.when(t == NS - 1)
        def _():
            pltpu.sync_copy(acc_sp.at[pl.ds((NS - 1) * RPT, LAST)],
                            out.at[pl.ds((NS - 1) * RPT, LAST)])

    @pl.when(c == 0)
    def _():
        one_side(ht0, out0)

    @pl.when(c == 1)
    def _():
        one_side(ht1, out1)


_agg_call = pl.kernel(
    _agg_body,
    out_type=[jax.ShapeDtypeStruct((N, HALF), jnp.float32)] * 2,
    mesh=_MESH,
    scratch_types=[
        pltpu.VMEM((2 * CH, EB), jnp.int32),
        pltpu.VMEM((2 * CH, EB), jnp.int32),
    ] + [pltpu.VMEM((EB, HALF), jnp.float32)] * NBUF
      + [pltpu.SemaphoreType.DMA] * NBUF
      + [pltpu.VMEM_SHARED((ACC_ROWS, HALF), jnp.float32)],
)


_RB = 2000


def _mm1a_body(x_ref, w_ref, h_ref):
    h_ref[...] = jnp.dot(x_ref[...], w_ref[...],
                         preferred_element_type=jnp.float32)


def _mm1b_body(h_ref, d0_ref, d1_ref, ht0_ref, ht1_ref, dis_ref):
    deg = d0_ref[:, :1] + d1_ref[:, :1] + 1.0
    dis = lax.rsqrt(deg)
    ht = h_ref[...] * dis
    ht0_ref[...] = ht[:, :HALF]
    ht1_ref[...] = ht[:, HALF:]
    dis_ref[...] = dis


def _mm2_body(a0_ref, a1_ref, dis_ref, b_ref, w_ref, ht0_ref, ht1_ref):
    acc = jnp.concatenate([a0_ref[...], a1_ref[...]], axis=1)
    dis = dis_ref[...]
    z = jnp.maximum(acc * dis + b_ref[...], 0.0)
    h = jnp.dot(z, w_ref[...], preferred_element_type=jnp.float32)
    ht = h * dis
    ht0_ref[...] = ht[:, :HALF]
    ht1_ref[...] = ht[:, HALF:]


def _fin_body(a0_ref, a1_ref, dis_ref, b_ref, o_ref):
    acc = jnp.concatenate([a0_ref[...], a1_ref[...]], axis=1)
    o_ref[...] = acc * dis_ref[...] + b_ref[...]


def _rows(shape):
    return pl.BlockSpec((_RB,) + shape[1:], lambda i: (i, 0))


def _whole(shape):
    return pl.BlockSpec(shape, lambda i: (0, 0))


_mm1a_call = pl.pallas_call(
    _mm1a_body,
    grid=(N // _RB,),
    in_specs=[_rows((N, D)), _whole((D, D))],
    out_specs=_rows((N, D)),
    out_shape=jax.ShapeDtypeStruct((N, D), jnp.float32),
)

_mm1b_call = pl.pallas_call(
    _mm1b_body,
    grid=(N // _RB,),
    in_specs=[_rows((N, D)), _rows((N, HALF)), _rows((N, HALF))],
    out_specs=[_rows((N, HALF)), _rows((N, HALF)), _rows((N, 1))],
    out_shape=[jax.ShapeDtypeStruct((N, HALF), jnp.float32),
               jax.ShapeDtypeStruct((N, HALF), jnp.float32),
               jax.ShapeDtypeStruct((N, 1), jnp.float32)],
)

_mm2_call = pl.pallas_call(
    _mm2_body,
    grid=(N // _RB,),
    in_specs=[_rows((N, HALF)), _rows((N, HALF)), _rows((N, 1)),
              _whole((1, D)), _whole((D, D))],
    out_specs=[_rows((N, HALF)), _rows((N, HALF))],
    out_shape=[jax.ShapeDtypeStruct((N, HALF), jnp.float32),
               jax.ShapeDtypeStruct((N, HALF), jnp.float32)],
)

_fin_call = pl.pallas_call(
    _fin_body,
    grid=(N // _RB,),
    in_specs=[_rows((N, HALF)), _rows((N, HALF)), _rows((N, 1)),
              _whole((1, D))],
    out_specs=_rows((N, D)),
    out_shape=jax.ShapeDtypeStruct((N, D), jnp.float32),
)


def kernel(x, edge_index, W1, b1, W2, b2):
    src = edge_index[0].astype(jnp.int32)
    dst = edge_index[1].astype(jnp.int32)
    npad = E_PAD - src.shape[0]
    src3 = jnp.concatenate(
        [src, jnp.zeros((npad,), jnp.int32)]).reshape(NS, NBT, EB)
    dst3 = jnp.concatenate(
        [dst, jnp.full((npad,), JUNK, jnp.int32)]).reshape(NS, NBT, EB)
    ones_in = jnp.ones((EB, HALF), jnp.float32)
    zeros_in = jnp.zeros((RPT, HALF), jnp.float32)

    h1 = _mm1a_call(x, W1)
    degp0, degp1 = _deg_call(dst3, ones_in, zeros_in)
    ht0, ht1, dis = _mm1b_call(h1, degp0, degp1)
    a10, a11 = _agg_call(src3, dst3, ht0, ht1)
    h20, h21 = _mm2_call(a10, a11, dis, b1.reshape(1, D), W2)
    a20, a21 = _agg_call(src3, dst3, h20, h21)
    return _fin_call(a20, a21, dis, b2.reshape(1, D))
